# baseline (speedup 1.0000x reference)
import functools

import jax
import jax.numpy as jnp
from jax.experimental import pallas as pl
from jax.experimental.pallas import tpu as pltpu

_N = 16384
_C = 1000
_BLK = 512
_GRID = _N // _BLK


def _ce_sum_block(x, tgt):
    m = jnp.max(x, axis=1, keepdims=True)
    s = jnp.sum(jnp.exp(x - m), axis=1)
    lse = jnp.log(s) + m[:, 0]
    cols = jax.lax.broadcasted_iota(jnp.int32, x.shape, 1)
    tl = jnp.sum(jnp.where(cols == tgt[:, None], x, 0.0), axis=1)
    return jnp.sum(lse - tl)


def _coteach_kernel(tgt_ref, l1_ref, l2_ref, out_ref):
    tgt = tgt_ref[...]
    s1 = _ce_sum_block(l1_ref[...], tgt)
    s2 = _ce_sum_block(l2_ref[...], tgt)
    out_ref[...] = jnp.stack([s1, s2]).reshape(1, 1, 2)


@jax.jit
def kernel(logits_1, logits_2, targets):
    tgt = targets.astype(jnp.int32)
    out = pl.pallas_call(
        _coteach_kernel,
        grid=(_GRID,),
        in_specs=[
            pl.BlockSpec((_BLK,), lambda i: (i,)),
            pl.BlockSpec((_BLK, _C), lambda i: (i, 0)),
            pl.BlockSpec((_BLK, _C), lambda i: (i, 0)),
        ],
        out_specs=pl.BlockSpec((1, 1, 2), lambda i: (i, 0, 0)),
        out_shape=jax.ShapeDtypeStruct((_GRID, 1, 2), jnp.float32),
        compiler_params=pltpu.CompilerParams(
            dimension_semantics=("parallel",),
        ),
    )(tgt, logits_1, logits_2)
    partial_sums = jnp.sum(out, axis=(0, 1)) * (1.0 / _N)
    return (partial_sums[0], partial_sums[1])


# speedup vs baseline: 1.0735x; 1.0735x over previous
import functools

import jax
import jax.numpy as jnp
from jax.experimental import pallas as pl
from jax.experimental.pallas import tpu as pltpu

_N = 16384
_C = 1000
_BLK = 2048
_GRID = _N // _BLK


def _ce_sum_block(x, tgt):
    m = jnp.max(x, axis=1, keepdims=True)
    s = jnp.sum(jnp.exp(x - m), axis=1)
    lse = jnp.log(s) + m[:, 0]
    cols = jax.lax.broadcasted_iota(jnp.int32, x.shape, 1)
    tl = jnp.sum(jnp.where(cols == tgt[:, None], x, 0.0), axis=1)
    return jnp.sum(lse - tl)


def _coteach_kernel(tgt_ref, l1_ref, l2_ref, out_ref):
    tgt = tgt_ref[...]
    s1 = _ce_sum_block(l1_ref[...], tgt)
    s2 = _ce_sum_block(l2_ref[...], tgt)
    out_ref[...] = jnp.stack([s1, s2]).reshape(1, 1, 2)


@jax.jit
def kernel(logits_1, logits_2, targets):
    tgt = targets.astype(jnp.int32)
    out = pl.pallas_call(
        _coteach_kernel,
        grid=(_GRID,),
        in_specs=[
            pl.BlockSpec((_BLK,), lambda i: (i,)),
            pl.BlockSpec((_BLK, _C), lambda i: (i, 0)),
            pl.BlockSpec((_BLK, _C), lambda i: (i, 0)),
        ],
        out_specs=pl.BlockSpec((1, 1, 2), lambda i: (i, 0, 0)),
        out_shape=jax.ShapeDtypeStruct((_GRID, 1, 2), jnp.float32),
        compiler_params=pltpu.CompilerParams(
            dimension_semantics=("parallel",),
        ),
    )(tgt, logits_1, logits_2)
    partial_sums = jnp.sum(out, axis=(0, 1)) * (1.0 / _N)
    return (partial_sums[0], partial_sums[1])
